# baseline (speedup 1.0000x reference)
import functools
import math

import jax
import jax.numpy as jnp
from jax.experimental import pallas as pl
from jax.experimental.pallas import tpu as pltpu

HPG = 4
CHUNK = 256


def _attn_kernel(x_ref, wqkv_ref, bqkv_ref, wo_ref, bo_ref, out_ref,
                 qkv_ref, ctx_ref, *, n, dph, scale):
    g = pl.program_id(1)
    q_cols = HPG * dph

    xb = x_ref[0]
    qkv_ref[...] = (jax.lax.dot_general(
        xb, wqkv_ref[0], (((1,), (0,)), ((), ())),
        preferred_element_type=jnp.float32) + bqkv_ref[0]
    ).astype(jnp.bfloat16)

    for h in range(HPG):
        def chunk_body(c, _, h=h):
            rows = pl.ds(c * CHUNK, CHUNK)
            qc = qkv_ref[rows, h * dph:(h + 1) * dph] * jnp.bfloat16(scale)
            k = qkv_ref[:, q_cols + h * dph:q_cols + (h + 1) * dph]
            v = qkv_ref[:, 2 * q_cols + h * dph:2 * q_cols + (h + 1) * dph]
            s = jax.lax.dot_general(
                qc, k, (((1,), (1,)), ((), ())),
                preferred_element_type=jnp.float32)
            m = jnp.max(s, axis=1, keepdims=True)
            e = jnp.exp(s - m)
            l = jnp.sum(e, axis=1, keepdims=True)
            cc = jnp.dot(e.astype(jnp.bfloat16), v,
                         preferred_element_type=jnp.float32) / l
            ctx_ref[rows, h * dph:(h + 1) * dph] = cc.astype(jnp.bfloat16)
            return 0

        jax.lax.fori_loop(0, n // CHUNK, chunk_body, 0)

    def oproj_body(c, _):
        rows = pl.ds(c * CHUNK, CHUNK)
        contrib = jnp.dot(ctx_ref[rows, :], wo_ref[0],
                          preferred_element_type=jnp.float32)

        @pl.when(g == 0)
        def _():
            out_ref[0, rows, :] = contrib + bo_ref[...]

        @pl.when(g != 0)
        def _():
            out_ref[0, rows, :] = out_ref[0, rows, :] + contrib

        return 0

    jax.lax.fori_loop(0, n // CHUNK, oproj_body, 0)


def kernel(x, Wq, bq, Wk, bk, Wv, bv, Wo, bo):
    B, N, D = x.shape
    H, _, DPH = Wq.shape
    G = H // HPG
    q_cols = HPG * DPH

    def group_w(W):
        return W.reshape(G, HPG, D, DPH).transpose(0, 2, 1, 3).reshape(
            G, D, q_cols)

    Wqkv = jnp.concatenate([group_w(Wq), group_w(Wk), group_w(Wv)],
                           axis=2).astype(jnp.bfloat16)
    bqkv = jnp.concatenate(
        [bq.reshape(G, 1, q_cols), bk.reshape(G, 1, q_cols),
         bv.reshape(G, 1, q_cols)], axis=2)
    Wog = Wo.reshape(G, q_cols, D).astype(jnp.bfloat16)
    bo2 = bo.reshape(1, D)
    xb16 = x.astype(jnp.bfloat16)

    body = functools.partial(_attn_kernel, n=N, dph=DPH,
                             scale=1.0 / math.sqrt(DPH))
    return pl.pallas_call(
        body,
        out_shape=jax.ShapeDtypeStruct((B, N, D), jnp.float32),
        grid=(B, G),
        in_specs=[
            pl.BlockSpec((1, N, D), lambda b, g: (b, 0, 0)),
            pl.BlockSpec((1, D, 3 * q_cols), lambda b, g: (g, 0, 0)),
            pl.BlockSpec((1, 1, 3 * q_cols), lambda b, g: (g, 0, 0)),
            pl.BlockSpec((1, q_cols, D), lambda b, g: (g, 0, 0)),
            pl.BlockSpec((1, D), lambda b, g: (0, 0)),
        ],
        out_specs=pl.BlockSpec((1, N, D), lambda b, g: (b, 0, 0)),
        scratch_shapes=[pltpu.VMEM((N, 3 * q_cols), jnp.bfloat16),
                        pltpu.VMEM((N, q_cols), jnp.bfloat16)],
        compiler_params=pltpu.CompilerParams(
            dimension_semantics=("parallel", "arbitrary"),
            vmem_limit_bytes=56 * 1024 * 1024,
        ),
        name="fused_mha",
    )(xb16, Wqkv, bqkv, Wog, bo2)


# speedup vs baseline: 1.2353x; 1.2353x over previous
import functools
import math

import jax
import jax.numpy as jnp
from jax.experimental import pallas as pl
from jax.experimental.pallas import tpu as pltpu

HPG = 2
CHUNK = 256


def _attn_kernel(x_ref, wqkv_ref, bqkv_ref, wo_ref, bo_ref, out_ref,
                 qkv_ref, ctx_ref, *, n, dph, scale):
    g = pl.program_id(1)
    q_cols = HPG * dph

    xb = x_ref[0]
    qkv_ref[...] = (jax.lax.dot_general(
        xb, wqkv_ref[0], (((1,), (0,)), ((), ())),
        preferred_element_type=jnp.float32) + bqkv_ref[0]
    ).astype(jnp.bfloat16)

    for h in range(HPG):
        k = qkv_ref[:, q_cols + h * dph:q_cols + (h + 1) * dph]
        v = qkv_ref[:, 2 * q_cols + h * dph:2 * q_cols + (h + 1) * dph]
        for c in range(n // CHUNK):
            rows = slice(c * CHUNK, (c + 1) * CHUNK)
            qc = qkv_ref[rows, h * dph:(h + 1) * dph] * jnp.bfloat16(scale)
            s = jax.lax.dot_general(
                qc, k, (((1,), (1,)), ((), ())),
                preferred_element_type=jnp.float32)
            m = jnp.max(s, axis=1, keepdims=True)
            e = jnp.exp(s - m)
            l = jnp.sum(e, axis=1, keepdims=True)
            cc = jnp.dot(e.astype(jnp.bfloat16), v,
                         preferred_element_type=jnp.float32) / l
            ctx_ref[rows, h * dph:(h + 1) * dph] = cc.astype(jnp.bfloat16)

    for c in range(n // CHUNK):
        rows = slice(c * CHUNK, (c + 1) * CHUNK)
        contrib = jnp.dot(ctx_ref[rows, :], wo_ref[0],
                          preferred_element_type=jnp.float32)

        @pl.when(g == 0)
        def _():
            out_ref[0, rows, :] = contrib + bo_ref[...]

        @pl.when(g != 0)
        def _():
            out_ref[0, rows, :] = out_ref[0, rows, :] + contrib


def kernel(x, Wq, bq, Wk, bk, Wv, bv, Wo, bo):
    B, N, D = x.shape
    H, _, DPH = Wq.shape
    G = H // HPG
    q_cols = HPG * DPH

    def group_w(W):
        return W.reshape(G, HPG, D, DPH).transpose(0, 2, 1, 3).reshape(
            G, D, q_cols)

    Wqkv = jnp.concatenate([group_w(Wq), group_w(Wk), group_w(Wv)],
                           axis=2).astype(jnp.bfloat16)
    bqkv = jnp.concatenate(
        [bq.reshape(G, 1, q_cols), bk.reshape(G, 1, q_cols),
         bv.reshape(G, 1, q_cols)], axis=2)
    Wog = Wo.reshape(G, q_cols, D).astype(jnp.bfloat16)
    bo2 = bo.reshape(1, D)
    xb16 = x.astype(jnp.bfloat16)

    body = functools.partial(_attn_kernel, n=N, dph=DPH,
                             scale=1.0 / math.sqrt(DPH))
    return pl.pallas_call(
        body,
        out_shape=jax.ShapeDtypeStruct((B, N, D), jnp.float32),
        grid=(B, G),
        in_specs=[
            pl.BlockSpec((1, N, D), lambda b, g: (b, 0, 0)),
            pl.BlockSpec((1, D, 3 * q_cols), lambda b, g: (g, 0, 0)),
            pl.BlockSpec((1, 1, 3 * q_cols), lambda b, g: (g, 0, 0)),
            pl.BlockSpec((1, q_cols, D), lambda b, g: (g, 0, 0)),
            pl.BlockSpec((1, D), lambda b, g: (0, 0)),
        ],
        out_specs=pl.BlockSpec((1, N, D), lambda b, g: (b, 0, 0)),
        scratch_shapes=[pltpu.VMEM((N, 3 * q_cols), jnp.bfloat16),
                        pltpu.VMEM((N, q_cols), jnp.bfloat16)],
        compiler_params=pltpu.CompilerParams(
            dimension_semantics=("parallel", "arbitrary"),
            vmem_limit_bytes=56 * 1024 * 1024,
        ),
        name="fused_mha",
    )(xb16, Wqkv, bqkv, Wog, bo2)


# speedup vs baseline: 1.6753x; 1.3562x over previous
import functools
import math

import jax
import jax.numpy as jnp
from jax.experimental import pallas as pl
from jax.experimental.pallas import tpu as pltpu

HPG = 4
CHUNK = 256


def _attn_kernel(x_ref, wqkv_ref, bqkv_ref, wo_ref, bo_ref, out_ref, ctx_ref,
                 *, n, dph, scale):
    g = pl.program_id(1)
    q_cols = HPG * dph

    xb = x_ref[0]
    qkv = jax.lax.dot_general(
        xb, wqkv_ref[0], (((1,), (0,)), ((), ())),
        preferred_element_type=jnp.float32) + bqkv_ref[0]

    for h in range(HPG):
        q = qkv[:, h * dph:(h + 1) * dph] * scale
        k = qkv[:, q_cols + h * dph:q_cols + (h + 1) * dph]
        v = qkv[:, 2 * q_cols + h * dph:2 * q_cols + (h + 1) * dph]
        for c in range(n // CHUNK):
            qc = q[c * CHUNK:(c + 1) * CHUNK]
            s = jax.lax.dot_general(
                qc, k, (((1,), (1,)), ((), ())),
                preferred_element_type=jnp.float32)
            m = jnp.max(s, axis=1, keepdims=True)
            e = jnp.exp(s - m)
            l = jnp.sum(e, axis=1, keepdims=True)
            cc = jnp.dot(e, v, preferred_element_type=jnp.float32) / l
            ctx_ref[c * CHUNK:(c + 1) * CHUNK, h * dph:(h + 1) * dph] = cc

    wo = wo_ref[0]
    for c in range(n // CHUNK):
        rows = slice(c * CHUNK, (c + 1) * CHUNK)
        contrib = jnp.dot(ctx_ref[rows, :], wo,
                          preferred_element_type=jnp.float32)

        @pl.when(g == 0)
        def _():
            out_ref[0, rows, :] = contrib + bo_ref[...]

        @pl.when(g != 0)
        def _():
            out_ref[0, rows, :] = out_ref[0, rows, :] + contrib


def kernel(x, Wq, bq, Wk, bk, Wv, bv, Wo, bo):
    B, N, D = x.shape
    H, _, DPH = Wq.shape
    G = H // HPG
    q_cols = HPG * DPH

    def group_w(W):
        return W.reshape(G, HPG, D, DPH).transpose(0, 2, 1, 3).reshape(
            G, D, q_cols)

    Wqkv = jnp.concatenate([group_w(Wq), group_w(Wk), group_w(Wv)],
                           axis=2)
    bqkv = jnp.concatenate(
        [bq.reshape(G, 1, q_cols), bk.reshape(G, 1, q_cols),
         bv.reshape(G, 1, q_cols)], axis=2)
    Wog = Wo.reshape(G, q_cols, D)
    bo2 = bo.reshape(1, D)

    body = functools.partial(_attn_kernel, n=N, dph=DPH,
                             scale=1.0 / math.sqrt(DPH))
    return pl.pallas_call(
        body,
        out_shape=jax.ShapeDtypeStruct((B, N, D), jnp.float32),
        grid=(B, G),
        in_specs=[
            pl.BlockSpec((1, N, D), lambda b, g: (b, 0, 0)),
            pl.BlockSpec((1, D, 3 * q_cols), lambda b, g: (g, 0, 0)),
            pl.BlockSpec((1, 1, 3 * q_cols), lambda b, g: (g, 0, 0)),
            pl.BlockSpec((1, q_cols, D), lambda b, g: (g, 0, 0)),
            pl.BlockSpec((1, D), lambda b, g: (0, 0)),
        ],
        out_specs=pl.BlockSpec((1, N, D), lambda b, g: (b, 0, 0)),
        scratch_shapes=[pltpu.VMEM((N, q_cols), jnp.float32)],
        compiler_params=pltpu.CompilerParams(
            dimension_semantics=("parallel", "arbitrary"),
            vmem_limit_bytes=56 * 1024 * 1024,
        ),
        name="fused_mha",
    )(x, Wqkv, bqkv, Wog, bo2)


# speedup vs baseline: 1.6942x; 1.0113x over previous
import functools
import math

import jax
import jax.numpy as jnp
from jax.experimental import pallas as pl
from jax.experimental.pallas import tpu as pltpu

HPG = 4
CHUNK = 256


def _attn_kernel(x_ref, wqkv_ref, bqkv_ref, wo_ref, bo_ref, out_ref, ctx_ref,
                 *, n, dph, scale):
    g = pl.program_id(1)
    q_cols = HPG * dph

    xb = x_ref[0]
    qkv = jax.lax.dot_general(
        xb, wqkv_ref[0], (((1,), (0,)), ((), ())),
        preferred_element_type=jnp.float32) + bqkv_ref[0]

    log2e_scale = scale * 1.4426950408889634
    for h in range(HPG):
        q = qkv[:, h * dph:(h + 1) * dph] * log2e_scale
        k = qkv[:, q_cols + h * dph:q_cols + (h + 1) * dph]
        v = qkv[:, 2 * q_cols + h * dph:2 * q_cols + (h + 1) * dph]
        for c in range(n // CHUNK):
            qc = q[c * CHUNK:(c + 1) * CHUNK]
            s = jax.lax.dot_general(
                qc, k, (((1,), (1,)), ((), ())),
                preferred_element_type=jnp.float32)
            m = jnp.max(s, axis=1, keepdims=True)
            e = jnp.exp2(s - m)
            l = jnp.sum(e, axis=1, keepdims=True)
            cc = jnp.dot(e, v, preferred_element_type=jnp.float32) / l
            ctx_ref[c * CHUNK:(c + 1) * CHUNK, h * dph:(h + 1) * dph] = cc

    wo = wo_ref[0]
    for c in range(n // CHUNK):
        rows = slice(c * CHUNK, (c + 1) * CHUNK)
        contrib = jnp.dot(ctx_ref[rows, :], wo,
                          preferred_element_type=jnp.float32)

        @pl.when(g == 0)
        def _():
            out_ref[0, rows, :] = contrib + bo_ref[...]

        @pl.when(g != 0)
        def _():
            out_ref[0, rows, :] = out_ref[0, rows, :] + contrib


def kernel(x, Wq, bq, Wk, bk, Wv, bv, Wo, bo):
    B, N, D = x.shape
    H, _, DPH = Wq.shape
    G = H // HPG
    q_cols = HPG * DPH

    def group_w(W):
        return W.reshape(G, HPG, D, DPH).transpose(0, 2, 1, 3).reshape(
            G, D, q_cols)

    Wqkv = jnp.concatenate([group_w(Wq), group_w(Wk), group_w(Wv)],
                           axis=2)
    bqkv = jnp.concatenate(
        [bq.reshape(G, 1, q_cols), bk.reshape(G, 1, q_cols),
         bv.reshape(G, 1, q_cols)], axis=2)
    Wog = Wo.reshape(G, q_cols, D)
    bo2 = bo.reshape(1, D)

    body = functools.partial(_attn_kernel, n=N, dph=DPH,
                             scale=1.0 / math.sqrt(DPH))
    return pl.pallas_call(
        body,
        out_shape=jax.ShapeDtypeStruct((B, N, D), jnp.float32),
        grid=(B, G),
        in_specs=[
            pl.BlockSpec((1, N, D), lambda b, g: (b, 0, 0)),
            pl.BlockSpec((1, D, 3 * q_cols), lambda b, g: (g, 0, 0)),
            pl.BlockSpec((1, 1, 3 * q_cols), lambda b, g: (g, 0, 0)),
            pl.BlockSpec((1, q_cols, D), lambda b, g: (g, 0, 0)),
            pl.BlockSpec((1, D), lambda b, g: (0, 0)),
        ],
        out_specs=pl.BlockSpec((1, N, D), lambda b, g: (b, 0, 0)),
        scratch_shapes=[pltpu.VMEM((N, q_cols), jnp.float32)],
        compiler_params=pltpu.CompilerParams(
            dimension_semantics=("parallel", "arbitrary"),
            vmem_limit_bytes=60 * 1024 * 1024,
        ),
        name="fused_mha",
    )(x, Wqkv, bqkv, Wog, bo2)


# speedup vs baseline: 1.7125x; 1.0108x over previous
import functools
import math

import jax
import jax.numpy as jnp
from jax.experimental import pallas as pl
from jax.experimental.pallas import tpu as pltpu

HPG = 4
CHUNK = 512


def _attn_kernel(x_ref, wqkv_ref, bqkv_ref, wo_ref, bo_ref, out_ref, ctx_ref,
                 *, n, dph, scale):
    g = pl.program_id(1)
    q_cols = HPG * dph

    xb = x_ref[0]
    qkv = jax.lax.dot_general(
        xb, wqkv_ref[0], (((1,), (0,)), ((), ())),
        preferred_element_type=jnp.float32) + bqkv_ref[0]

    log2e_scale = scale * 1.4426950408889634
    for h in range(HPG):
        q = (qkv[:, h * dph:(h + 1) * dph] * log2e_scale).astype(jnp.bfloat16)
        k = qkv[:, q_cols + h * dph:q_cols + (h + 1) * dph].astype(jnp.bfloat16)
        v = qkv[:, 2 * q_cols + h * dph:2 * q_cols + (h + 1) * dph]
        for c in range(n // CHUNK):
            qc = q[c * CHUNK:(c + 1) * CHUNK]
            s = jax.lax.dot_general(
                qc, k, (((1,), (1,)), ((), ())),
                preferred_element_type=jnp.float32)
            m = jnp.max(s, axis=1, keepdims=True)
            e = jnp.exp2(s - m)
            l = jnp.sum(e, axis=1, keepdims=True)
            cc = jnp.dot(e, v, preferred_element_type=jnp.float32) / l
            ctx_ref[c * CHUNK:(c + 1) * CHUNK, h * dph:(h + 1) * dph] = cc

    wo = wo_ref[0]
    for c in range(n // CHUNK):
        rows = slice(c * CHUNK, (c + 1) * CHUNK)
        contrib = jnp.dot(ctx_ref[rows, :], wo,
                          preferred_element_type=jnp.float32)

        @pl.when(g == 0)
        def _():
            out_ref[0, rows, :] = contrib + bo_ref[...]

        @pl.when(g != 0)
        def _():
            out_ref[0, rows, :] = out_ref[0, rows, :] + contrib


def kernel(x, Wq, bq, Wk, bk, Wv, bv, Wo, bo):
    B, N, D = x.shape
    H, _, DPH = Wq.shape
    G = H // HPG
    q_cols = HPG * DPH

    def group_w(W):
        return W.reshape(G, HPG, D, DPH).transpose(0, 2, 1, 3).reshape(
            G, D, q_cols)

    Wqkv = jnp.concatenate([group_w(Wq), group_w(Wk), group_w(Wv)],
                           axis=2).astype(jnp.bfloat16)
    bqkv = jnp.concatenate(
        [bq.reshape(G, 1, q_cols), bk.reshape(G, 1, q_cols),
         bv.reshape(G, 1, q_cols)], axis=2)
    Wog = Wo.reshape(G, q_cols, D)
    bo2 = bo.reshape(1, D)

    body = functools.partial(_attn_kernel, n=N, dph=DPH,
                             scale=1.0 / math.sqrt(DPH))
    return pl.pallas_call(
        body,
        out_shape=jax.ShapeDtypeStruct((B, N, D), jnp.float32),
        grid=(B, G),
        in_specs=[
            pl.BlockSpec((1, N, D), lambda b, g: (b, 0, 0)),
            pl.BlockSpec((1, D, 3 * q_cols), lambda b, g: (g, 0, 0)),
            pl.BlockSpec((1, 1, 3 * q_cols), lambda b, g: (g, 0, 0)),
            pl.BlockSpec((1, q_cols, D), lambda b, g: (g, 0, 0)),
            pl.BlockSpec((1, D), lambda b, g: (0, 0)),
        ],
        out_specs=pl.BlockSpec((1, N, D), lambda b, g: (b, 0, 0)),
        scratch_shapes=[pltpu.VMEM((N, q_cols), jnp.float32)],
        compiler_params=pltpu.CompilerParams(
            dimension_semantics=("parallel", "arbitrary"),
            vmem_limit_bytes=60 * 1024 * 1024,
        ),
        name="fused_mha",
    )(x.astype(jnp.bfloat16), Wqkv, bqkv, Wog, bo2)
